# baseline (speedup 1.0000x reference)
import functools

import jax
import jax.numpy as jnp
from jax import lax
from jax.experimental import pallas as pl
from jax.experimental.pallas import tpu as pltpu
from jax.experimental.pallas import tpu_sc as plsc

B = 16
C = 256
HH = 38
PW = HH + 2
NPOS = HH * HH
K2 = 9
ROWS_PER_BATCH = NPOS * K2
NC, NS = 2, 16
NW = NC * NS
ROWS_TOTAL = B * ROWS_PER_BATCH
CH = 96
NCHUNK = ROWS_TOTAL // CH
ROUNDS = -(-NCHUNK // NW)
IDX_PAD = ROUNDS * CH


def _tc_transpose(x2):
    def body(x_ref, o_ref):
        o_ref[0] = x_ref[0].T

    return pl.pallas_call(
        body,
        grid=(B,),
        in_specs=[pl.BlockSpec((1, C, NPOS), lambda b: (b, 0, 0))],
        out_specs=pl.BlockSpec((1, NPOS, C), lambda b: (b, 0, 0)),
        out_shape=jax.ShapeDtypeStruct((B, NPOS, C), jnp.float32),
    )(x2)


def _sc_gather(table, idx_all):
    mesh = plsc.VectorSubcoreMesh(core_axis_name="c", subcore_axis_name="s",
                                  num_cores=NC, num_subcores=NS)

    @functools.partial(
        pl.kernel,
        out_type=jax.ShapeDtypeStruct((ROWS_TOTAL, C), jnp.float32),
        mesh=mesh,
        scratch_types=[
            pltpu.VMEM((IDX_PAD,), jnp.int32),
            pltpu.VMEM((CH, C), jnp.float32),
            pltpu.SemaphoreType.DMA,
        ],
    )
    def k(table_hbm, idx_hbm, out_hbm, idx_v, buf, sem):
        wid = lax.axis_index("s") * NC + lax.axis_index("c")
        ibase = pl.multiple_of(wid * IDX_PAD, 128)
        pltpu.sync_copy(idx_hbm.at[pl.ds(ibase, IDX_PAD)], idx_v)

        def body(j, carry):
            c = j * NW + wid

            @pl.when(c < NCHUNK)
            def _():
                pltpu.async_copy(
                    table_hbm.at[idx_v.at[pl.ds(j * CH, CH)]], buf, sem).wait()
                obase = pl.multiple_of(c * CH, 8)
                pltpu.sync_copy(buf, out_hbm.at[pl.ds(obase, CH)])

            return carry

        lax.fori_loop(0, ROUNDS, body, 0)

    return k(table, idx_all)


def _build_idx(counts):
    cnt = counts.astype(jnp.int32)
    gi = cnt // PW
    gj = cnt % PW
    src = jnp.clip(gi - 1, 0, HH - 1) * HH + jnp.clip(gj - 1, 0, HH - 1)
    flat = src.reshape(-1)
    boffs = (jnp.arange(B, dtype=jnp.int32) * NPOS)[:, None]
    glob = (flat[None, :] + boffs).reshape(-1)
    padded = jnp.pad(glob, (0, ROUNDS * NW * CH - ROWS_TOTAL))
    per_w = padded.reshape(ROUNDS, NW, CH).transpose(1, 0, 2)
    return per_w.reshape(NW * IDX_PAD)


def kernel(fm, counts):
    x2 = fm.reshape(B, C, NPOS)
    xt = _tc_transpose(x2)
    idx_all = _build_idx(counts)
    out = _sc_gather(xt.reshape(B * NPOS, C), idx_all)
    return out.reshape(B, NPOS, K2 * C)
